# Initial kernel scaffold; baseline (speedup 1.0000x reference)
import jax
import jax.numpy as jnp
from jax.experimental import pallas as pl


def kernel(loc_preds, loc_targets, conf_preds, conf_targets):
    raise NotImplementedError("write your pallas kernel here")


# speedup vs baseline: 15.5356x; 15.5356x over previous
import dataclasses
import functools

import jax
import jax.numpy as jnp
from jax import lax
from jax.experimental import pallas as pl
from jax.experimental.pallas import tpu as pltpu
from jax.experimental.pallas import tpu_sc as plsc

_B = 32
_N = 21824
_NP = 21888
_NB = 2432
_LANES = 16
_NVREG = _NP // _LANES


def _tc_body(n_total, lp_ref, lt_ref, cp_ref, y_ref, m_ref, stats_ref):
    pid = pl.program_id(0)

    d = lp_ref[...] - lt_ref[...]
    ad = jnp.abs(d)
    sl1 = jnp.where(ad < 1.0, 0.5 * d * d, ad - 0.5)
    s4 = sl1[0:32] + sl1[32:64] + sl1[64:96] + sl1[96:128]

    y = y_ref[...]
    pos = y > 0
    posf = pos.astype(jnp.float32)

    x0 = cp_ref[0:32]
    x1 = cp_ref[32:64]
    mx = jnp.maximum(x0, x1)
    mn = jnp.minimum(x0, x1)
    lse = mx + jnp.log(1.0 + jnp.exp(mn - mx))
    ce = lse - jnp.where(pos, x1, x0)

    col = lax.broadcasted_iota(jnp.int32, ce.shape, 1) + pid * ce.shape[1]
    valid = col < n_total
    m = jnp.where(pos | (~valid), 0.0, jnp.maximum(ce, 0.0))
    m_ref[...] = m

    npos_r = jnp.sum(posf, axis=1, keepdims=True)
    ce_r = jnp.sum(ce * posf, axis=1, keepdims=True)
    loc_r = jnp.sum(s4 * posf, axis=1, keepdims=True)
    lane = lax.broadcasted_iota(jnp.int32, (32, 128), 1)
    upd = (jnp.where(lane == 0, npos_r, 0.0)
           + jnp.where(lane == 1, ce_r, 0.0)
           + jnp.where(lane == 2, loc_r, 0.0))

    @pl.when(pid == 0)
    def _():
        stats_ref[...] = jnp.zeros_like(stats_ref)

    stats_ref[...] += upd


def _sc_topk_body(m_hbm, kk_hbm, out_hbm, row_v, k_v, out_v):
    wid = lax.axis_index("s") * 2 + lax.axis_index("c")
    pltpu.sync_copy(m_hbm.at[wid], row_v)
    pltpu.sync_copy(kk_hbm.at[wid], k_v)
    k = jnp.max(k_v[...])

    def pass_a(i, carry):
        s, c = carry
        v = row_v[pl.ds(i * _LANES, _LANES)]
        return s + v, c + (v > 0.0).astype(jnp.int32)

    s0 = jnp.zeros((_LANES,), jnp.float32)
    c0 = jnp.zeros((_LANES,), jnp.int32)
    svec, cvec = lax.fori_loop(0, _NVREG, pass_a, (s0, c0))
    sum0 = jnp.sum(svec)
    count0 = jnp.sum(cvec)

    def fast(_):
        return sum0

    def slow(_):
        def bis(_, lohi):
            lo, hi = lohi
            mid = lo + (hi - lo + 1) // 2

            def cb(i, c):
                key = plsc.bitcast(row_v[pl.ds(i * _LANES, _LANES)], jnp.int32)
                return c + (key >= mid).astype(jnp.int32)

            cnt = jnp.sum(lax.fori_loop(0, _NVREG, cb, c0))
            take = cnt >= k
            return (jnp.where(take, mid, lo), jnp.where(take, hi, mid - 1))

        tkey, _hi = lax.fori_loop(0, 31, bis, (jnp.int32(0),
                                               jnp.int32(0x7F800000)))
        tval = jnp.max(plsc.bitcast(jnp.full((_LANES,), tkey, jnp.int32),
                                    jnp.float32))

        def fb(i, carry):
            sg, cg = carry
            v = row_v[pl.ds(i * _LANES, _LANES)]
            gt = plsc.bitcast(v, jnp.int32) > tkey
            return (sg + jnp.where(gt, v, 0.0), cg + gt.astype(jnp.int32))

        sgv, cgv = lax.fori_loop(0, _NVREG, fb, (s0, c0))
        sum_gt = jnp.sum(sgv)
        cnt_gt = jnp.sum(cgv)
        return sum_gt + (k - cnt_gt).astype(jnp.float32) * tval

    res = lax.cond(k >= count0, fast, slow, None)
    out_v[...] = jnp.full((_LANES,), res, jnp.float32)
    pltpu.sync_copy(out_v, out_hbm.at[wid])


def _tc_stage(lp_t, lt_t, cp_t, y_p):
    return pl.pallas_call(
        functools.partial(_tc_body, _N),
        grid=(_NP // _NB,),
        in_specs=[
            pl.BlockSpec((128, _NB), lambda i: (0, i)),
            pl.BlockSpec((128, _NB), lambda i: (0, i)),
            pl.BlockSpec((64, _NB), lambda i: (0, i)),
            pl.BlockSpec((32, _NB), lambda i: (0, i)),
        ],
        out_specs=[
            pl.BlockSpec((32, _NB), lambda i: (0, i)),
            pl.BlockSpec((32, 128), lambda i: (0, 0)),
        ],
        out_shape=[
            jax.ShapeDtypeStruct((_B, _NP), jnp.float32),
            jax.ShapeDtypeStruct((_B, 128), jnp.float32),
        ],
    )(lp_t, lt_t, cp_t, y_p)


def _sc_stage(m, kk):
    mesh = plsc.VectorSubcoreMesh(core_axis_name="c", subcore_axis_name="s")
    cp = pltpu.CompilerParams()
    if "needs_layout_passes" in pltpu.CompilerParams.__dataclass_fields__:
        cp = dataclasses.replace(cp, needs_layout_passes=False)
    fn = pl.kernel(
        _sc_topk_body,
        out_type=jax.ShapeDtypeStruct((_B, _LANES), jnp.float32),
        mesh=mesh,
        compiler_params=cp,
        scratch_types=[
            pltpu.VMEM((_NP,), jnp.float32),
            pltpu.VMEM((_LANES,), jnp.int32),
            pltpu.VMEM((_LANES,), jnp.float32),
        ],
    )
    return fn(m, kk)


def kernel(loc_preds, loc_targets, conf_preds, conf_targets):
    B, N = conf_targets.shape
    pad = _NP - N

    lp_t = jnp.pad(jnp.transpose(loc_preds, (2, 0, 1)).reshape(4 * B, N),
                   ((0, 0), (0, pad)))
    lt_t = jnp.pad(jnp.transpose(loc_targets, (2, 0, 1)).reshape(4 * B, N),
                   ((0, 0), (0, pad)))
    cp_t = jnp.pad(jnp.transpose(conf_preds, (2, 0, 1)).reshape(2 * B, N),
                   ((0, 0), (0, pad)))
    y_p = jnp.pad(conf_targets.astype(jnp.int32), ((0, 0), (0, pad)))

    m, stats = _tc_stage(lp_t, lt_t, cp_t, y_p)

    num_pos = stats[:, 0]
    pos_ce = stats[:, 1]
    loc_s = stats[:, 2]
    num_matched = jnp.sum(num_pos)

    k = jnp.minimum(3 * num_pos.astype(jnp.int32), N - 1)
    kk = jnp.broadcast_to(k[:, None], (B, _LANES))

    topk = _sc_stage(m, kk)

    return (jnp.sum(loc_s) + jnp.sum(pos_ce) + jnp.sum(topk[:, 0])) / num_matched
